# Initial kernel scaffold; baseline (speedup 1.0000x reference)
import functools

import jax
import jax.numpy as jnp
from jax import lax
from jax.experimental import pallas as pl
from jax.experimental.pallas import tpu as pltpu


def _knn_mean_body(k_ref, x_ref, y_ref, preds_ref, out_ref):
    x = x_ref[0]
    y = y_ref[0]
    k = k_ref[0, 0]

    g = lax.dot_general(x, y, (((1,), (1,)), ((), ())),
                        preferred_element_type=jnp.float32)
    x2 = jnp.sum(x * x, axis=1, keepdims=True)
    y2 = jnp.sum(y * y, axis=1, keepdims=True).T
    sq = jnp.maximum(x2 + y2 - 2.0 * g, 0.0)

    bits = jnp.maximum(lax.bitcast_convert_type(sq, jnp.int32), 0)

    r = bits.shape[0]
    lo0 = jnp.zeros((r, 1), jnp.int32)
    hi0 = jnp.full((r, 1), jnp.int32(0x7F800000))

    def step(_, carry):
        lo, hi = carry
        mid = lo + (hi - lo) // 2
        cnt = jnp.sum((bits <= mid).astype(jnp.int32), axis=1, keepdims=True)
        ge = cnt >= k
        return jnp.where(ge, lo, mid + 1), jnp.where(ge, mid, hi)

    lo, hi = lax.fori_loop(0, 31, step, (lo0, hi0))

    sel = (bits <= hi).astype(jnp.float32)
    acc = lax.dot_general(sel, preds_ref[0], (((1,), (0,)), ((), ())),
                          preferred_element_type=jnp.float32)
    out_ref[0] = acc / k.astype(jnp.float32)


@functools.partial(jax.jit, static_argnames=("interpret",))
def kernel(points, preds, k_vector, interpret=False):
    B, N, D = points.shape
    c = preds.shape[2]
    R = 256

    k = (jnp.argmax(jax.nn.softmax(k_vector, axis=0), axis=0) + 1).astype(jnp.int32)
    k_arr = k.reshape(1, 1)

    grid = (B, N // R)
    out = pl.pallas_call(
        _knn_mean_body,
        grid=grid,
        in_specs=[
            pl.BlockSpec(memory_space=pltpu.SMEM),
            pl.BlockSpec((1, R, D), lambda b, i: (b, i, 0)),
            pl.BlockSpec((1, N, D), lambda b, i: (b, 0, 0)),
            pl.BlockSpec((1, N, c), lambda b, i: (b, 0, 0)),
        ],
        out_specs=pl.BlockSpec((1, R, c), lambda b, i: (b, i, 0)),
        out_shape=jax.ShapeDtypeStruct((B, N, c), jnp.float32),
        interpret=interpret,
    )(k_arr, points, points, preds)
    return out


# speedup vs baseline: 8.9520x; 8.9520x over previous
import functools

import jax
import jax.numpy as jnp
from jax import lax
from jax.experimental import pallas as pl
from jax.experimental.pallas import tpu as pltpu


def _knn_mean_body(k_ref, x_ref, y_ref, x2_ref, y2t_ref, preds_ref, out_ref):
    x = x_ref[0]
    y = y_ref[0]
    k = k_ref[0, 0]

    g = lax.dot_general(x, y, (((1,), (1,)), ((), ())),
                        preferred_element_type=jnp.float32)
    sq = jnp.maximum(x2_ref[0] + y2t_ref[0] - 2.0 * g, 0.0)
    d = jnp.sqrt(sq)

    bits = jnp.maximum(lax.bitcast_convert_type(d, jnp.int32), 0)

    r = bits.shape[0]
    lo0 = jnp.zeros((r, 1), jnp.int32)
    hi0 = jnp.full((r, 1), jnp.int32(0x7F800000))

    def step(_, carry):
        lo, hi = carry
        mid = lo + (hi - lo) // 2
        cnt = jnp.sum((bits <= mid).astype(jnp.int32), axis=1, keepdims=True)
        ge = cnt >= k
        return jnp.where(ge, lo, mid + 1), jnp.where(ge, mid, hi)

    _, t = lax.fori_loop(0, 31, step, (lo0, hi0))

    lt = bits < t
    eq = bits == t
    need = k - jnp.sum(lt.astype(jnp.int32), axis=1, keepdims=True)
    n = bits.shape[1]
    idx = lax.broadcasted_iota(jnp.int32, (r, n), 1)

    def step2(_, carry):
        lo, hi = carry
        mid = lo + (hi - lo) // 2
        cnt = jnp.sum((eq & (idx <= mid)).astype(jnp.int32), axis=1,
                      keepdims=True)
        ge = cnt >= need
        return jnp.where(ge, lo, mid + 1), jnp.where(ge, mid, hi)

    _, m = lax.fori_loop(0, 11, step2, (jnp.zeros((r, 1), jnp.int32),
                                        jnp.full((r, 1), jnp.int32(n - 1))))

    sel = (lt | (eq & (idx <= m))).astype(jnp.float32)
    acc = lax.dot_general(sel, preds_ref[0], (((1,), (0,)), ((), ())),
                          preferred_element_type=jnp.float32,
                          precision=lax.Precision.HIGHEST)
    out_ref[0] = acc / k.astype(jnp.float32)


@functools.partial(jax.jit, static_argnames=("interpret",))
def kernel(points, preds, k_vector, interpret=False):
    B, N, D = points.shape
    c = preds.shape[2]
    R = 256

    k = (jnp.argmax(jax.nn.softmax(k_vector, axis=0), axis=0) + 1).astype(jnp.int32)
    k_arr = k.reshape(1, 1)

    x2 = jnp.sum(points * points, axis=-1, keepdims=True)
    y2t = jnp.swapaxes(x2, -1, -2)

    out = pl.pallas_call(
        _knn_mean_body,
        grid=(B, N // R),
        in_specs=[
            pl.BlockSpec(memory_space=pltpu.SMEM),
            pl.BlockSpec((1, R, D), lambda b, i: (b, i, 0)),
            pl.BlockSpec((1, N, D), lambda b, i: (b, 0, 0)),
            pl.BlockSpec((1, R, 1), lambda b, i: (b, i, 0)),
            pl.BlockSpec((1, 1, N), lambda b, i: (b, 0, 0)),
            pl.BlockSpec((1, N, c), lambda b, i: (b, 0, 0)),
        ],
        out_specs=pl.BlockSpec((1, R, c), lambda b, i: (b, i, 0)),
        out_shape=jax.ShapeDtypeStruct((B, N, c), jnp.float32),
        interpret=interpret,
    )(k_arr, points, points, x2, y2t, preds)
    return out


# speedup vs baseline: 11.9590x; 1.3359x over previous
import functools

import jax
import jax.numpy as jnp
from jax import lax
from jax.experimental import pallas as pl
from jax.experimental.pallas import tpu as pltpu


def _knn_mean_body(k_ref, x_ref, y_ref, x2_ref, y2t_ref, preds_ref, out_ref):
    x = x_ref[0]
    y = y_ref[0]
    k = k_ref[0, 0]

    g = lax.dot_general(x, y, (((1,), (1,)), ((), ())),
                        preferred_element_type=jnp.float32)
    sq = jnp.maximum(x2_ref[0] + y2t_ref[0] - 2.0 * g, 0.0)
    d = jnp.sqrt(sq)

    bits = jnp.maximum(lax.bitcast_convert_type(d, jnp.int32), 0)

    r, n = bits.shape
    lo0 = jnp.zeros((r, 1), jnp.int32)
    hi0 = jnp.full((r, 1), jnp.int32(0x7F800000))
    tf0 = jnp.full((r, 1), jnp.int32(-1))

    def w_cond(carry):
        i, lo, hi, tf = carry
        return (i < 31) & jnp.any((tf < 0) & (lo < hi))

    def w_body(carry):
        i, lo, hi, tf = carry
        mid = lo + (hi - lo) // 2
        cnt = jnp.sum((bits <= mid).astype(jnp.int32), axis=1, keepdims=True)
        ge = cnt >= k
        tf = jnp.where((cnt == k) & (tf < 0), mid, tf)
        return (i + 1, jnp.where(ge, lo, mid + 1), jnp.where(ge, mid, hi), tf)

    _, lo, hi, tf = lax.while_loop(w_cond, w_body, (jnp.int32(0), lo0, hi0, tf0))
    t = jnp.where(tf >= 0, tf, hi)

    lt = bits < t
    eq = bits == t
    idx = lax.broadcasted_iota(jnp.int32, (r, n), 1)

    def tie_fn():
        need = k - jnp.sum(lt.astype(jnp.int32), axis=1, keepdims=True)

        def step2(_, carry):
            lo2, hi2 = carry
            mid = lo2 + (hi2 - lo2) // 2
            cnt = jnp.sum((eq & (idx <= mid)).astype(jnp.int32), axis=1,
                          keepdims=True)
            ge = cnt >= need
            return jnp.where(ge, lo2, mid + 1), jnp.where(ge, mid, hi2)

        _, m2 = lax.fori_loop(0, 11, step2, (jnp.zeros((r, 1), jnp.int32),
                                             jnp.full((r, 1), jnp.int32(n - 1))))
        return m2

    m = lax.cond(jnp.any(tf < 0), tie_fn,
                 lambda: jnp.full((r, 1), jnp.int32(n - 1)))

    sel = (lt | (eq & (idx <= m))).astype(jnp.float32)
    acc = lax.dot_general(sel, preds_ref[0], (((1,), (0,)), ((), ())),
                          preferred_element_type=jnp.float32,
                          precision=lax.Precision.HIGHEST)
    out_ref[0] = acc / k.astype(jnp.float32)


@functools.partial(jax.jit, static_argnames=("interpret",))
def kernel(points, preds, k_vector, interpret=False):
    B, N, D = points.shape
    c = preds.shape[2]
    R = 256

    k = (jnp.argmax(jax.nn.softmax(k_vector, axis=0), axis=0) + 1).astype(jnp.int32)
    k_arr = k.reshape(1, 1)

    x2 = jnp.sum(points * points, axis=-1, keepdims=True)
    y2t = jnp.swapaxes(x2, -1, -2)

    out = pl.pallas_call(
        _knn_mean_body,
        grid=(B, N // R),
        in_specs=[
            pl.BlockSpec(memory_space=pltpu.SMEM),
            pl.BlockSpec((1, R, D), lambda b, i: (b, i, 0)),
            pl.BlockSpec((1, N, D), lambda b, i: (b, 0, 0)),
            pl.BlockSpec((1, R, 1), lambda b, i: (b, i, 0)),
            pl.BlockSpec((1, 1, N), lambda b, i: (b, 0, 0)),
            pl.BlockSpec((1, N, c), lambda b, i: (b, 0, 0)),
        ],
        out_specs=pl.BlockSpec((1, R, c), lambda b, i: (b, i, 0)),
        out_shape=jax.ShapeDtypeStruct((B, N, c), jnp.float32),
        interpret=interpret,
    )(k_arr, points, points, x2, y2t, preds)
    return out
